# baseline (speedup 1.0000x reference)
import functools

import jax
import jax.numpy as jnp
from jax import lax
from jax.experimental import pallas as pl
from jax.experimental.pallas import tpu as pltpu
from jax.experimental.pallas import tpu_sc as plsc

N = 10000
E = 160000
IN_DIM = 128
H1 = 32
H2 = 16

NC, NS = 2, 16
NW = NC * NS
NP = 10240
RPT = NP // NS
EPW = E // NW
CHUNK = 128
NFULL = EPW // CHUNK
TAIL = EPW - NFULL * CHUNK
DEGW = 8

_sc_mesh = functools.partial(
    plsc.VectorSubcoreMesh, core_axis_name="c", subcore_axis_name="s")


DBUF = 6


@functools.partial(
    pl.kernel,
    out_type=[jax.ShapeDtypeStruct((NC, NP), jnp.float32),
              jax.ShapeDtypeStruct((NC, NP), jnp.float32)],
    mesh=_sc_mesh(),
    compiler_params=pltpu.CompilerParams(use_tc_tiling_on_sc=False,
                                         needs_layout_passes=False),
    scratch_types=[
        [pltpu.VMEM((CHUNK,), jnp.int32)] * DBUF,
        [pltpu.VMEM((CHUNK,), jnp.int32)] * DBUF,
        [pltpu.SemaphoreType.DMA] * DBUF,
        [pltpu.SemaphoreType.DMA] * DBUF,
        pltpu.VMEM((TAIL,), jnp.int32),
        pltpu.VMEM((TAIL,), jnp.int32),
        pltpu.VMEM((CHUNK, DEGW), jnp.float32),
        pltpu.VMEM((RPT, DEGW), jnp.float32),
        pltpu.VMEM((RPT,), jnp.float32),
        pltpu.VMEM_SHARED((NP, DEGW), jnp.float32),
        pltpu.VMEM_SHARED((NP, DEGW), jnp.float32),
    ],
)
def _sc_degrees(ei_hbm, ones_hbm, zeros_hbm, dego_hbm, degi_hbm,
                sidx, didx, semi, sems, sidx_t, didx_t, ones_v, stage,
                stage1, acc_o, acc_i):
    cid = lax.axis_index("c")
    sid = lax.axis_index("s")
    base = (cid * NS + sid) * EPW

    pltpu.sync_copy(zeros_hbm, stage)
    pltpu.sync_copy(stage, acc_o.at[pl.ds(sid * RPT, RPT)])
    pltpu.sync_copy(stage, acc_i.at[pl.ds(sid * RPT, RPT)])
    pltpu.sync_copy(ones_hbm, ones_v)
    plsc.subcore_barrier()

    def off(c):
        return jnp.minimum(base + c * CHUNK, E - CHUNK)

    def issue_idx(b, c):
        pltpu.async_copy(ei_hbm.at[0, pl.ds(off(c), CHUNK)], sidx[b], semi[b])
        pltpu.async_copy(ei_hbm.at[1, pl.ds(off(c), CHUNK)], didx[b], semi[b])

    def wait_idx(b):
        pltpu.make_async_copy(ei_hbm.at[0, pl.ds(0, CHUNK)], sidx[b],
                              semi[b]).wait()
        pltpu.make_async_copy(ei_hbm.at[1, pl.ds(0, CHUNK)], didx[b],
                              semi[b]).wait()

    def issue_scatters(b):
        pltpu.async_copy(ones_v, acc_o.at[sidx[b]], sems[b], add=True)
        pltpu.async_copy(ones_v, acc_i.at[didx[b]], sems[b], add=True)

    def wait_scatters(b):
        pltpu.make_async_copy(ones_v, acc_o.at[sidx[b]], sems[b]).wait()
        pltpu.make_async_copy(ones_v, acc_i.at[didx[b]], sems[b]).wait()

    def slot(c):
        return c % DBUF

    def step(c, x, wait_sc):
        if wait_sc:
            wait_scatters((x + 4) % DBUF)
        issue_idx((x + 4) % DBUF, c + 4)
        wait_idx((x + 2) % DBUF)
        issue_scatters(x)

    for b in range(4):
        issue_idx(b, b)
    wait_idx(0)
    wait_idx(1)
    step(0, 0, False)
    step(1, 1, False)

    def body(k, carry):
        c0 = 2 + k * DBUF
        for i in range(DBUF):
            step(c0 + i, (2 + i) % DBUF, True)
        return carry

    lax.fori_loop(0, (NFULL - 3) // DBUF, body, 0)
    step(NFULL - 1, slot(NFULL - 1), True)

    wait_idx(slot(NFULL + 2))
    wait_idx(slot(NFULL + 3))
    wait_scatters(slot(NFULL - 2))
    wait_scatters(slot(NFULL - 1))

    offt = base + NFULL * CHUNK
    pltpu.sync_copy(ei_hbm.at[0, pl.ds(offt, TAIL)], sidx_t)
    pltpu.sync_copy(ei_hbm.at[1, pl.ds(offt, TAIL)], didx_t)
    pltpu.sync_copy(ones_v.at[pl.ds(0, TAIL)], acc_o.at[sidx_t], add=True)
    pltpu.sync_copy(ones_v.at[pl.ds(0, TAIL)], acc_i.at[didx_t], add=True)
    plsc.subcore_barrier()

    sl = pl.ds(sid * RPT, RPT)
    col0 = jnp.zeros((16,), jnp.int32)
    lanes = lax.broadcasted_iota(jnp.int32, (16,), 0)

    def extract(acc, out_hbm):
        pltpu.sync_copy(acc.at[sl], stage)

        def xb(i, carry):
            stage1[pl.ds(i * 16, 16)] = plsc.load_gather(
                stage, [lanes + i * 16, col0])
            return carry

        lax.fori_loop(0, RPT // 16, xb, 0)
        pltpu.sync_copy(stage1, out_hbm.at[cid, sl])

    extract(acc_o, dego_hbm)
    extract(acc_i, degi_hbm)


NBUF = 6


@functools.partial(
    pl.kernel,
    out_type=jax.ShapeDtypeStruct((NC, NP, H1), jnp.float32),
    mesh=_sc_mesh(),
    compiler_params=pltpu.CompilerParams(use_tc_tiling_on_sc=False),
    scratch_types=[
        [pltpu.VMEM((CHUNK,), jnp.int32)] * NBUF,
        [pltpu.VMEM((CHUNK,), jnp.int32)] * NBUF,
        [pltpu.VMEM((CHUNK, H1), jnp.float32)] * NBUF,
        [pltpu.SemaphoreType.DMA] * NBUF,
        [pltpu.SemaphoreType.DMA] * NBUF,
        [pltpu.SemaphoreType.DMA] * NBUF,
        pltpu.VMEM((TAIL,), jnp.int32),
        pltpu.VMEM((TAIL,), jnp.int32),
        pltpu.VMEM((TAIL, H1), jnp.float32),
        pltpu.VMEM((RPT, H1), jnp.float32),
        pltpu.VMEM_SHARED((NP, H1), jnp.float32),
        pltpu.SemaphoreType.DMA,
    ],
)
def _sc_segsum(ei_hbm, ys_hbm, zeros_hbm, out_hbm,
               sidx, didx, rows, semi, semg, sems,
               sidx_t, didx_t, rows_t, stage, acc, sem):
    cid = lax.axis_index("c")
    sid = lax.axis_index("s")
    base = (cid * NS + sid) * EPW

    pltpu.sync_copy(zeros_hbm, stage)
    pltpu.sync_copy(stage, acc.at[pl.ds(sid * RPT, RPT)])
    plsc.subcore_barrier()

    def off(c):
        return jnp.minimum(base + c * CHUNK, E - CHUNK)

    def issue_idx(b, c):
        pltpu.async_copy(ei_hbm.at[0, pl.ds(off(c), CHUNK)], sidx[b], semi[b])
        pltpu.async_copy(ei_hbm.at[1, pl.ds(off(c), CHUNK)], didx[b], semi[b])

    def wait_idx(b):
        pltpu.make_async_copy(ei_hbm.at[0, pl.ds(0, CHUNK)], sidx[b],
                              semi[b]).wait()
        pltpu.make_async_copy(ei_hbm.at[1, pl.ds(0, CHUNK)], didx[b],
                              semi[b]).wait()

    def issue_gather(b):
        pltpu.async_copy(ys_hbm.at[sidx[b]], rows[b], semg[b])

    def wait_gather(b):
        pltpu.make_async_copy(ys_hbm.at[sidx[b]], rows[b], semg[b]).wait()

    def issue_scatter(b):
        pltpu.async_copy(rows[b], acc.at[didx[b]], sems[b], add=True)

    def wait_scatter(b):
        pltpu.make_async_copy(rows[b], acc.at[didx[b]], sems[b]).wait()

    def slot(c):
        return c % NBUF

    def step(c, x, wait_sc):
        if wait_sc:
            wait_scatter((x + 4) % NBUF)
        issue_idx((x + 4) % NBUF, c + 4)
        wait_idx((x + 2) % NBUF)
        issue_gather((x + 2) % NBUF)
        wait_gather(x)
        issue_scatter(x)

    for b in range(4):
        issue_idx(b, b)
    wait_idx(0)
    issue_gather(0)
    wait_idx(1)
    issue_gather(1)
    step(0, 0, False)
    step(1, 1, False)

    def body(k, carry):
        c0 = 2 + k * NBUF
        for i in range(NBUF):
            step(c0 + i, (2 + i) % NBUF, True)
        return carry

    lax.fori_loop(0, (NFULL - 3) // NBUF, body, 0)
    step(NFULL - 1, slot(NFULL - 1), True)

    wait_idx(slot(NFULL + 2))
    wait_idx(slot(NFULL + 3))
    wait_gather(slot(NFULL))
    wait_gather(slot(NFULL + 1))
    wait_scatter(slot(NFULL - 2))
    wait_scatter(slot(NFULL - 1))

    offt = base + NFULL * CHUNK
    pltpu.sync_copy(ei_hbm.at[0, pl.ds(offt, TAIL)], sidx_t)
    pltpu.sync_copy(ei_hbm.at[1, pl.ds(offt, TAIL)], didx_t)
    pltpu.async_copy(ys_hbm.at[sidx_t], rows_t, sem).wait()
    pltpu.sync_copy(rows_t, acc.at[didx_t], add=True)
    plsc.subcore_barrier()

    sl = pl.ds(sid * RPT, RPT)
    pltpu.sync_copy(acc.at[sl], stage)
    pltpu.sync_copy(stage, out_hbm.at[cid, sl])


def _scale_from_partials(p):
    deg = p[0] + p[1]
    s = lax.rsqrt(jnp.maximum(deg, 1.0))
    return s[:, None]


def _tc1a_body(x_ref, w0_ref, y_ref):
    y_ref[...] = jnp.dot(x_ref[...], w0_ref[...],
                         preferred_element_type=jnp.float32)


def _tc1b_body(y_ref, dego_ref, ys_ref):
    ys_ref[...] = y_ref[...] * _scale_from_partials(dego_ref[...])


def _tc2_body(s0_ref, degi_ref, dego_ref, b0_ref, hs_ref):
    agg = s0_ref[0] + s0_ref[1]
    h = jnp.maximum(agg * _scale_from_partials(degi_ref[...]) + b0_ref[...],
                    0.0)
    hs_ref[...] = h * _scale_from_partials(dego_ref[...])


def _tc3_body(s1_ref, degi_ref, w1_ref, b1_ref, w2_ref, b2_ref, noise_ref,
              z_ref, zt_ref):
    agg = (s1_ref[0] + s1_ref[1]) * _scale_from_partials(degi_ref[...])
    mean = jnp.dot(agg, w1_ref[...],
                   preferred_element_type=jnp.float32) + b1_ref[...]
    log_std = jnp.dot(agg, w2_ref[...],
                      preferred_element_type=jnp.float32) + b2_ref[...]
    z = mean + noise_ref[...] * jnp.exp(log_std)
    z_ref[...] = z
    zt_ref[...] = z.T


RB = 1280
BM = 400


def _dec_body(zi_ref, zj_ref, o_ref):
    logits = jnp.dot(zi_ref[...], zj_ref[...],
                     preferred_element_type=jnp.float32)
    o_ref[...] = 1.0 / (1.0 + jnp.exp(-logits))


def _noise_expr():
    return jax.random.normal(jax.random.key(42), (N, H2), dtype=jnp.float32)


try:
    import numpy as _np
    _NOISE = _np.asarray(_noise_expr())
except Exception:
    _NOISE = None


def kernel(features, edge_index, W0, b0, W1, b1, W2, b2):
    zeros_deg = jnp.zeros((RPT, DEGW), jnp.float32)
    ones_deg = jnp.ones((CHUNK, DEGW), jnp.float32)
    zeros_h1 = jnp.zeros((RPT, H1), jnp.float32)

    dego, degi = _sc_degrees(edge_index, ones_deg, zeros_deg)

    gridr = (pl.cdiv(N, RB),)

    y0 = pl.pallas_call(
        _tc1a_body,
        grid=gridr,
        in_specs=[
            pl.BlockSpec((RB, IN_DIM), lambda i: (i, 0)),
            pl.BlockSpec((IN_DIM, H1), lambda i: (0, 0)),
        ],
        out_specs=pl.BlockSpec((RB, H1), lambda i: (i, 0)),
        out_shape=jax.ShapeDtypeStruct((N, H1), jnp.float32),
    )(features, W0)

    y0s = pl.pallas_call(
        _tc1b_body,
        grid=gridr,
        in_specs=[
            pl.BlockSpec((RB, H1), lambda i: (i, 0)),
            pl.BlockSpec((NC, RB), lambda i: (0, i)),
        ],
        out_specs=pl.BlockSpec((RB, H1), lambda i: (i, 0)),
        out_shape=jax.ShapeDtypeStruct((N, H1), jnp.float32),
    )(y0, dego)

    s0 = _sc_segsum(edge_index, y0s, zeros_h1)

    hs = pl.pallas_call(
        _tc2_body,
        grid=gridr,
        in_specs=[
            pl.BlockSpec((NC, RB, H1), lambda i: (0, i, 0)),
            pl.BlockSpec((NC, RB), lambda i: (0, i)),
            pl.BlockSpec((NC, RB), lambda i: (0, i)),
            pl.BlockSpec((1, H1), lambda i: (0, 0)),
        ],
        out_specs=pl.BlockSpec((RB, H1), lambda i: (i, 0)),
        out_shape=jax.ShapeDtypeStruct((N, H1), jnp.float32),
    )(s0, degi, dego, b0.reshape(1, H1))

    s1 = _sc_segsum(edge_index, hs, zeros_h1)

    noise = _NOISE if _NOISE is not None else _noise_expr()
    z, zt = pl.pallas_call(
        _tc3_body,
        grid=gridr,
        in_specs=[
            pl.BlockSpec((NC, RB, H1), lambda i: (0, i, 0)),
            pl.BlockSpec((NC, RB), lambda i: (0, i)),
            pl.BlockSpec((H1, H2), lambda i: (0, 0)),
            pl.BlockSpec((1, H2), lambda i: (0, 0)),
            pl.BlockSpec((H1, H2), lambda i: (0, 0)),
            pl.BlockSpec((1, H2), lambda i: (0, 0)),
            pl.BlockSpec((RB, H2), lambda i: (i, 0)),
        ],
        out_specs=[
            pl.BlockSpec((RB, H2), lambda i: (i, 0)),
            pl.BlockSpec((H2, RB), lambda i: (0, i)),
        ],
        out_shape=[jax.ShapeDtypeStruct((N, H2), jnp.float32),
                   jax.ShapeDtypeStruct((H2, N), jnp.float32)],
    )(s1, degi, W1, b1.reshape(1, H2), W2, b2.reshape(1, H2), noise)

    adj = pl.pallas_call(
        _dec_body,
        grid=(N // BM,),
        in_specs=[
            pl.BlockSpec((BM, H2), lambda i: (i, 0)),
            pl.BlockSpec((H2, N), lambda i: (0, 0)),
        ],
        out_specs=pl.BlockSpec((BM, N), lambda i: (i, 0)),
        out_shape=jax.ShapeDtypeStruct((N, N), jnp.float32),
    )(z, zt)
    return adj


# speedup vs baseline: 1.0177x; 1.0177x over previous
import functools

import jax
import jax.numpy as jnp
from jax import lax
from jax.experimental import pallas as pl
from jax.experimental.pallas import tpu as pltpu
from jax.experimental.pallas import tpu_sc as plsc

N = 10000
E = 160000
IN_DIM = 128
H1 = 32
H2 = 16

NC, NS = 2, 16
NW = NC * NS
NP = 10240
RPT = NP // NS
EPW = E // NW
CHUNK = 128
NFULL = EPW // CHUNK
TAIL = EPW - NFULL * CHUNK
DEGW = 8

_sc_mesh = functools.partial(
    plsc.VectorSubcoreMesh, core_axis_name="c", subcore_axis_name="s")


DBUF = 6


@functools.partial(
    pl.kernel,
    out_type=[jax.ShapeDtypeStruct((NC, NP), jnp.float32),
              jax.ShapeDtypeStruct((NC, NP), jnp.float32)],
    mesh=_sc_mesh(),
    compiler_params=pltpu.CompilerParams(use_tc_tiling_on_sc=False,
                                         needs_layout_passes=False),
    scratch_types=[
        [pltpu.VMEM((CHUNK,), jnp.int32)] * DBUF,
        [pltpu.VMEM((CHUNK,), jnp.int32)] * DBUF,
        [pltpu.SemaphoreType.DMA] * DBUF,
        [pltpu.SemaphoreType.DMA] * DBUF,
        pltpu.VMEM((TAIL,), jnp.int32),
        pltpu.VMEM((TAIL,), jnp.int32),
        pltpu.VMEM((CHUNK, DEGW), jnp.float32),
        pltpu.VMEM((RPT, DEGW), jnp.float32),
        pltpu.VMEM((RPT,), jnp.float32),
        pltpu.VMEM_SHARED((NP, DEGW), jnp.float32),
        pltpu.VMEM_SHARED((NP, DEGW), jnp.float32),
    ],
)
def _sc_degrees(ei_hbm, ones_hbm, zeros_hbm, dego_hbm, degi_hbm,
                sidx, didx, semi, sems, sidx_t, didx_t, ones_v, stage,
                stage1, acc_o, acc_i):
    cid = lax.axis_index("c")
    sid = lax.axis_index("s")
    base = (cid * NS + sid) * EPW

    pltpu.sync_copy(zeros_hbm, stage)
    pltpu.sync_copy(stage, acc_o.at[pl.ds(sid * RPT, RPT)])
    pltpu.sync_copy(stage, acc_i.at[pl.ds(sid * RPT, RPT)])
    pltpu.sync_copy(ones_hbm, ones_v)
    plsc.subcore_barrier()

    def off(c):
        return jnp.minimum(base + c * CHUNK, E - CHUNK)

    def issue_idx(b, c):
        pltpu.async_copy(ei_hbm.at[0, pl.ds(off(c), CHUNK)], sidx[b], semi[b])
        pltpu.async_copy(ei_hbm.at[1, pl.ds(off(c), CHUNK)], didx[b], semi[b])

    def wait_idx(b):
        pltpu.make_async_copy(ei_hbm.at[0, pl.ds(0, CHUNK)], sidx[b],
                              semi[b]).wait()
        pltpu.make_async_copy(ei_hbm.at[1, pl.ds(0, CHUNK)], didx[b],
                              semi[b]).wait()

    def issue_scatters(b):
        pltpu.async_copy(ones_v, acc_o.at[sidx[b]], sems[b], add=True)
        pltpu.async_copy(ones_v, acc_i.at[didx[b]], sems[b], add=True)

    def wait_scatters(b):
        pltpu.make_async_copy(ones_v, acc_o.at[sidx[b]], sems[b]).wait()
        pltpu.make_async_copy(ones_v, acc_i.at[didx[b]], sems[b]).wait()

    def slot(c):
        return c % DBUF

    def step(c, x, wait_sc):
        if wait_sc:
            wait_scatters((x + 4) % DBUF)
        issue_idx((x + 4) % DBUF, c + 4)
        wait_idx((x + 2) % DBUF)
        issue_scatters(x)

    for b in range(4):
        issue_idx(b, b)
    wait_idx(0)
    wait_idx(1)
    step(0, 0, False)
    step(1, 1, False)

    def body(k, carry):
        c0 = 2 + k * DBUF
        for i in range(DBUF):
            step(c0 + i, (2 + i) % DBUF, True)
        return carry

    lax.fori_loop(0, (NFULL - 3) // DBUF, body, 0)
    step(NFULL - 1, slot(NFULL - 1), True)

    wait_idx(slot(NFULL + 2))
    wait_idx(slot(NFULL + 3))
    wait_scatters(slot(NFULL - 2))
    wait_scatters(slot(NFULL - 1))

    offt = base + NFULL * CHUNK
    pltpu.sync_copy(ei_hbm.at[0, pl.ds(offt, TAIL)], sidx_t)
    pltpu.sync_copy(ei_hbm.at[1, pl.ds(offt, TAIL)], didx_t)
    pltpu.sync_copy(ones_v.at[pl.ds(0, TAIL)], acc_o.at[sidx_t], add=True)
    pltpu.sync_copy(ones_v.at[pl.ds(0, TAIL)], acc_i.at[didx_t], add=True)
    plsc.subcore_barrier()

    sl = pl.ds(sid * RPT, RPT)
    col0 = jnp.zeros((16,), jnp.int32)
    lanes = lax.broadcasted_iota(jnp.int32, (16,), 0)

    def extract(acc, out_hbm):
        pltpu.sync_copy(acc.at[sl], stage)

        def xb(i, carry):
            stage1[pl.ds(i * 16, 16)] = plsc.load_gather(
                stage, [lanes + i * 16, col0])
            return carry

        lax.fori_loop(0, RPT // 16, xb, 0)
        pltpu.sync_copy(stage1, out_hbm.at[cid, sl])

    extract(acc_o, dego_hbm)
    extract(acc_i, degi_hbm)


NBUF = 6


@functools.partial(
    pl.kernel,
    out_type=jax.ShapeDtypeStruct((NC, NP, H1), jnp.float32),
    mesh=_sc_mesh(),
    compiler_params=pltpu.CompilerParams(use_tc_tiling_on_sc=False),
    scratch_types=[
        [pltpu.VMEM((CHUNK,), jnp.int32)] * NBUF,
        [pltpu.VMEM((CHUNK,), jnp.int32)] * NBUF,
        [pltpu.VMEM((CHUNK, H1), jnp.float32)] * NBUF,
        [pltpu.SemaphoreType.DMA] * NBUF,
        [pltpu.SemaphoreType.DMA] * NBUF,
        [pltpu.SemaphoreType.DMA] * NBUF,
        pltpu.VMEM((TAIL,), jnp.int32),
        pltpu.VMEM((TAIL,), jnp.int32),
        pltpu.VMEM((TAIL, H1), jnp.float32),
        pltpu.VMEM((RPT, H1), jnp.float32),
        pltpu.VMEM_SHARED((NP, H1), jnp.float32),
        pltpu.SemaphoreType.DMA,
    ],
)
def _sc_segsum(ei_hbm, ys_hbm, zeros_hbm, out_hbm,
               sidx, didx, rows, semi, semg, sems,
               sidx_t, didx_t, rows_t, stage, acc, sem):
    cid = lax.axis_index("c")
    sid = lax.axis_index("s")
    base = (cid * NS + sid) * EPW

    pltpu.sync_copy(zeros_hbm, stage)
    pltpu.sync_copy(stage, acc.at[pl.ds(sid * RPT, RPT)])
    plsc.subcore_barrier()

    def off(c):
        return jnp.minimum(base + c * CHUNK, E - CHUNK)

    def issue_idx(b, c):
        pltpu.async_copy(ei_hbm.at[0, pl.ds(off(c), CHUNK)], sidx[b], semi[b])
        pltpu.async_copy(ei_hbm.at[1, pl.ds(off(c), CHUNK)], didx[b], semi[b])

    def wait_idx(b):
        pltpu.make_async_copy(ei_hbm.at[0, pl.ds(0, CHUNK)], sidx[b],
                              semi[b]).wait()
        pltpu.make_async_copy(ei_hbm.at[1, pl.ds(0, CHUNK)], didx[b],
                              semi[b]).wait()

    def issue_gather(b):
        pltpu.async_copy(ys_hbm.at[sidx[b]], rows[b], semg[b])

    def wait_gather(b):
        pltpu.make_async_copy(ys_hbm.at[sidx[b]], rows[b], semg[b]).wait()

    def issue_scatter(b):
        pltpu.async_copy(rows[b], acc.at[didx[b]], sems[b], add=True)

    def wait_scatter(b):
        pltpu.make_async_copy(rows[b], acc.at[didx[b]], sems[b]).wait()

    def slot(c):
        return c % NBUF

    def step(c, x, wait_sc):
        if wait_sc:
            wait_scatter((x + 4) % NBUF)
        issue_idx((x + 4) % NBUF, c + 4)
        wait_idx((x + 2) % NBUF)
        issue_gather((x + 2) % NBUF)
        wait_gather(x)
        issue_scatter(x)

    for b in range(4):
        issue_idx(b, b)
    wait_idx(0)
    issue_gather(0)
    wait_idx(1)
    issue_gather(1)
    step(0, 0, False)
    step(1, 1, False)

    def body(k, carry):
        c0 = 2 + k * NBUF
        for i in range(NBUF):
            step(c0 + i, (2 + i) % NBUF, True)
        return carry

    lax.fori_loop(0, (NFULL - 3) // NBUF, body, 0)
    step(NFULL - 1, slot(NFULL - 1), True)

    wait_idx(slot(NFULL + 2))
    wait_idx(slot(NFULL + 3))
    wait_gather(slot(NFULL))
    wait_gather(slot(NFULL + 1))
    wait_scatter(slot(NFULL - 2))
    wait_scatter(slot(NFULL - 1))

    offt = base + NFULL * CHUNK
    pltpu.sync_copy(ei_hbm.at[0, pl.ds(offt, TAIL)], sidx_t)
    pltpu.sync_copy(ei_hbm.at[1, pl.ds(offt, TAIL)], didx_t)
    pltpu.async_copy(ys_hbm.at[sidx_t], rows_t, sem).wait()
    pltpu.sync_copy(rows_t, acc.at[didx_t], add=True)
    plsc.subcore_barrier()

    sl = pl.ds(sid * RPT, RPT)
    pltpu.sync_copy(acc.at[sl], stage)
    pltpu.sync_copy(stage, out_hbm.at[cid, sl])


def _scale_from_partials(p, n):
    deg = p[0] + p[1]
    s = lax.rsqrt(jnp.maximum(deg, 1.0))
    return s[:n][:, None]


def _tc1a_body(x_ref, w0_ref, y_ref):
    y_ref[...] = jnp.dot(x_ref[...], w0_ref[...],
                         preferred_element_type=jnp.float32)


def _tc1b_body(y_ref, dego_ref, ys_ref):
    ys_ref[...] = y_ref[...] * _scale_from_partials(dego_ref[...], N)


def _tc2_body(s0_ref, degi_ref, dego_ref, b0_ref, hs_ref):
    agg = s0_ref[0, :N, :] + s0_ref[1, :N, :]
    h = jnp.maximum(
        agg * _scale_from_partials(degi_ref[...], N) + b0_ref[...], 0.0)
    hs_ref[...] = h * _scale_from_partials(dego_ref[...], N)


def _tc3_body(s1_ref, degi_ref, w1_ref, b1_ref, w2_ref, b2_ref, noise_ref,
              z_ref, zt_ref):
    agg = (s1_ref[0, :N, :] + s1_ref[1, :N, :]) * _scale_from_partials(
        degi_ref[...], N)
    mean = jnp.dot(agg, w1_ref[...],
                   preferred_element_type=jnp.float32) + b1_ref[...]
    log_std = jnp.dot(agg, w2_ref[...],
                      preferred_element_type=jnp.float32) + b2_ref[...]
    z = mean + noise_ref[...] * jnp.exp(log_std)
    z_ref[...] = z
    zt_ref[...] = z.T


BM = 400


def _dec_body(zi_ref, zj_ref, o_ref):
    logits = jnp.dot(zi_ref[...], zj_ref[...],
                     preferred_element_type=jnp.float32)
    o_ref[...] = 1.0 / (1.0 + jnp.exp(-logits))


def _noise_expr():
    return jax.random.normal(jax.random.key(42), (N, H2), dtype=jnp.float32)


try:
    import numpy as _np
    _NOISE = _np.asarray(_noise_expr())
except Exception:
    _NOISE = None


def kernel(features, edge_index, W0, b0, W1, b1, W2, b2):
    zeros_deg = jnp.zeros((RPT, DEGW), jnp.float32)
    ones_deg = jnp.ones((CHUNK, DEGW), jnp.float32)
    zeros_h1 = jnp.zeros((RPT, H1), jnp.float32)

    dego, degi = _sc_degrees(edge_index, ones_deg, zeros_deg)

    y0 = pl.pallas_call(
        _tc1a_body,
        out_shape=jax.ShapeDtypeStruct((N, H1), jnp.float32),
    )(features, W0)

    y0s = pl.pallas_call(
        _tc1b_body,
        out_shape=jax.ShapeDtypeStruct((N, H1), jnp.float32),
    )(y0, dego)

    s0 = _sc_segsum(edge_index, y0s, zeros_h1)

    hs = pl.pallas_call(
        _tc2_body,
        out_shape=jax.ShapeDtypeStruct((N, H1), jnp.float32),
    )(s0, degi, dego, b0.reshape(1, H1))

    s1 = _sc_segsum(edge_index, hs, zeros_h1)

    noise = _NOISE if _NOISE is not None else _noise_expr()
    z, zt = pl.pallas_call(
        _tc3_body,
        out_shape=[jax.ShapeDtypeStruct((N, H2), jnp.float32),
                   jax.ShapeDtypeStruct((H2, N), jnp.float32)],
    )(s1, degi, W1, b1.reshape(1, H2), W2, b2.reshape(1, H2), noise)

    adj = pl.pallas_call(
        _dec_body,
        grid=(N // BM,),
        in_specs=[
            pl.BlockSpec((BM, H2), lambda i: (i, 0)),
            pl.BlockSpec((H2, N), lambda i: (0, 0)),
        ],
        out_specs=pl.BlockSpec((BM, N), lambda i: (i, 0)),
        out_shape=jax.ShapeDtypeStruct((N, N), jnp.float32),
    )(z, zt)
    return adj
